# baseline (speedup 1.0000x reference)
import functools

import jax
import jax.numpy as jnp
from jax import lax
from jax.experimental import pallas as pl
from jax.experimental.pallas import tpu as pltpu
from jax.experimental.pallas import tpu_sc as plsc

DIM = 32
LANES = 16
NUM_CORES = 2
NUM_SUBCORES = 16
NUM_WORKERS = NUM_CORES * NUM_SUBCORES
BATCH = 16384
ROWS = 1000001


def _build(batch):
    b_per_w = batch // NUM_WORKERS
    n_ent = b_per_w * DIM
    mesh = plsc.VectorSubcoreMesh(core_axis_name="c", subcore_axis_name="s")

    @functools.partial(
        pl.kernel,
        mesh=mesh,
        compiler_params=pltpu.CompilerParams(needs_layout_passes=False,
                                             use_tc_tiling_on_sc=False),
        out_type=jax.ShapeDtypeStruct((batch,), jnp.float32),
        scratch_types=[
            pltpu.VMEM((b_per_w,), jnp.int32),
            pltpu.VMEM((b_per_w,), jnp.int32),
            pltpu.VMEM((n_ent,), jnp.int32),
            pltpu.VMEM((n_ent,), jnp.int32),
            pltpu.VMEM((n_ent,), jnp.float32),
            pltpu.VMEM((n_ent,), jnp.float32),
            pltpu.VMEM((b_per_w,), jnp.float32),
            pltpu.VMEM((b_per_w,), jnp.float32),
            pltpu.VMEM((LANES,), jnp.float32),
            pltpu.VMEM((b_per_w,), jnp.float32),
            pltpu.SemaphoreType.DMA,
            pltpu.SemaphoreType.DMA,
            pltpu.SemaphoreType.DMA,
            pltpu.SemaphoreType.DMA,
        ],
    )
    def mf(user_hbm, item_hbm, uflat_hbm, iflat_hbm, ubias_hbm, ibias_hbm,
           gbias_hbm, out_hbm,
           uidx_v, iidx_v, uw_v, iw_v, uval_v, ival_v, ub_v, ib_v, gb_v,
           out_v, sem_u, sem_i, sem_ub, sem_ib):
        wid = lax.axis_index("s") * NUM_CORES + lax.axis_index("c")
        base = wid * b_per_w
        n_chunks = b_per_w // LANES

        pltpu.sync_copy(user_hbm.at[pl.ds(base, b_per_w)], uidx_v)
        pltpu.sync_copy(item_hbm.at[pl.ds(base, b_per_w)], iidx_v)
        pltpu.sync_copy(gbias_hbm, gb_v)

        def offs_body(idx_v, w_v):
            def body(g, carry):
                r = idx_v[pl.ds(g * LANES, LANES)]
                for c in range(DIM):
                    w_v[pl.ds(c * b_per_w + g * LANES, LANES)] = r + c * ROWS
                return carry
            lax.fori_loop(0, n_chunks, body, 0)

        offs_body(uidx_v, uw_v)
        cu = pltpu.async_copy(uflat_hbm.at[uw_v], uval_v, sem_u)
        cub = pltpu.async_copy(ubias_hbm.at[uidx_v], ub_v, sem_ub)

        offs_body(iidx_v, iw_v)
        ci = pltpu.async_copy(iflat_hbm.at[iw_v], ival_v, sem_i)
        cib = pltpu.async_copy(ibias_hbm.at[iidx_v], ib_v, sem_ib)

        cu.wait()
        ci.wait()
        cub.wait()
        cib.wait()
        gb = gb_v[...]

        def dot_body(g, carry):
            o = g * LANES
            acc = uval_v[pl.ds(o, LANES)] * ival_v[pl.ds(o, LANES)]
            for c in range(1, DIM):
                oc = c * b_per_w + o
                acc = acc + (uval_v[pl.ds(oc, LANES)]
                             * ival_v[pl.ds(oc, LANES)])
            x = acc + ub_v[pl.ds(o, LANES)] + ib_v[pl.ds(o, LANES)] + gb
            out_v[pl.ds(o, LANES)] = 1.0 / (1.0 + jnp.exp(-x))
            return carry

        lax.fori_loop(0, n_chunks, dot_body, 0)

        pltpu.sync_copy(out_v, out_hbm.at[pl.ds(base, b_per_w)])

    return mf


_MF = _build(BATCH)


def kernel(user, item, user_embedding, item_embedding, user_bias, item_bias,
           bias):
    u = user.astype(jnp.int32)
    it = item.astype(jnp.int32)
    uflat = user_embedding.T.reshape(-1)
    iflat = item_embedding.T.reshape(-1)
    ub = user_bias.reshape(-1)
    ib = item_bias.reshape(-1)
    gb = jnp.broadcast_to(bias.astype(jnp.float32), (LANES,))
    return _MF(u, it, uflat, iflat, ub, ib, gb)


# speedup vs baseline: 5.7748x; 5.7748x over previous
import functools

import jax
import jax.numpy as jnp
from jax import lax
from jax.experimental import pallas as pl
from jax.experimental.pallas import tpu as pltpu
from jax.experimental.pallas import tpu_sc as plsc

DIM = 32
LANES = 16
NUM_CORES = 2
NUM_SUBCORES = 16
NUM_WORKERS = NUM_CORES * NUM_SUBCORES
BATCH = 16384


def _build(batch):
    b_per_w = batch // NUM_WORKERS
    mesh = plsc.VectorSubcoreMesh(core_axis_name="c", subcore_axis_name="s")

    @functools.partial(
        pl.kernel,
        mesh=mesh,
        compiler_params=pltpu.CompilerParams(needs_layout_passes=False,
                                             use_tc_tiling_on_sc=False),
        out_type=jax.ShapeDtypeStruct((batch,), jnp.float32),
        scratch_types=[
            pltpu.VMEM((b_per_w,), jnp.int32),
            pltpu.VMEM((b_per_w,), jnp.int32),
            pltpu.VMEM((b_per_w, DIM), jnp.float32),
            pltpu.VMEM((b_per_w, DIM), jnp.float32),
            pltpu.VMEM((b_per_w,), jnp.float32),
            pltpu.VMEM((b_per_w,), jnp.float32),
            pltpu.VMEM((LANES,), jnp.float32),
            pltpu.VMEM((b_per_w,), jnp.float32),
            pltpu.VMEM((b_per_w,), jnp.float32),
            pltpu.SemaphoreType.DMA,
            pltpu.SemaphoreType.DMA,
            pltpu.SemaphoreType.DMA,
            pltpu.SemaphoreType.DMA,
        ],
    )
    def mf(user_hbm, item_hbm, uemb_hbm, iemb_hbm, ubias_hbm, ibias_hbm,
           gbias_hbm, out_hbm,
           uidx_v, iidx_v, urows_v, irows_v, ub_v, ib_v, gb_v, dots_v, out_v,
           sem_u, sem_i, sem_ub, sem_ib):
        wid = lax.axis_index("s") * NUM_CORES + lax.axis_index("c")
        base = wid * b_per_w

        pltpu.sync_copy(user_hbm.at[pl.ds(base, b_per_w)], uidx_v)
        pltpu.sync_copy(item_hbm.at[pl.ds(base, b_per_w)], iidx_v)
        pltpu.sync_copy(gbias_hbm, gb_v)

        cu = pltpu.async_copy(uemb_hbm.at[uidx_v], urows_v, sem_u)
        ci = pltpu.async_copy(iemb_hbm.at[iidx_v], irows_v, sem_i)
        cub = pltpu.async_copy(ubias_hbm.at[uidx_v], ub_v, sem_ub)
        cib = pltpu.async_copy(ibias_hbm.at[iidx_v], ib_v, sem_ib)
        cu.wait()
        ci.wait()

        lane = lax.iota(jnp.int32, LANES)
        last_lane = lane == (LANES - 1)

        def dot_body(b, carry):
            u0 = urows_v[b, pl.ds(0, LANES)]
            u1 = urows_v[b, pl.ds(LANES, LANES)]
            i0 = irows_v[b, pl.ds(0, LANES)]
            i1 = irows_v[b, pl.ds(LANES, LANES)]
            p = u0 * i0 + u1 * i1
            cum = plsc.cumsum(p)
            plsc.store_scatter(dots_v, [jnp.full((LANES,), b, jnp.int32)],
                               cum, mask=last_lane)
            return carry

        lax.fori_loop(0, b_per_w, dot_body, 0, unroll=8)

        cub.wait()
        cib.wait()
        gb = gb_v[...]

        def sig_body(g, carry):
            o = g * LANES
            x = (dots_v[pl.ds(o, LANES)] + ub_v[pl.ds(o, LANES)]
                 + ib_v[pl.ds(o, LANES)] + gb)
            out_v[pl.ds(o, LANES)] = 1.0 / (1.0 + jnp.exp(-x))
            return carry

        lax.fori_loop(0, b_per_w // LANES, sig_body, 0)

        pltpu.sync_copy(out_v, out_hbm.at[pl.ds(base, b_per_w)])

    return mf


_MF = _build(BATCH)


def kernel(user, item, user_embedding, item_embedding, user_bias, item_bias,
           bias):
    u = user.astype(jnp.int32)
    it = item.astype(jnp.int32)
    ub = user_bias.reshape(-1)
    ib = item_bias.reshape(-1)
    gb = jnp.broadcast_to(bias.astype(jnp.float32), (LANES,))
    return _MF(u, it, user_embedding, item_embedding, ub, ib, gb)
